# Initial kernel scaffold; baseline (speedup 1.0000x reference)
import jax
import jax.numpy as jnp
from jax.experimental import pallas as pl


def kernel(tokens, task_ids, task_embed, gate_W, gate_b, We, be, Wu, bu):
    raise NotImplementedError("write your pallas kernel here")


# speedup vs baseline: 6.2469x; 6.2469x over previous
import functools

import jax
import jax.numpy as jnp
from jax.experimental import pallas as pl
from jax.experimental.pallas import tpu as pltpu


def _gelu_exact(x):
    return 0.5 * x * (1.0 + jax.lax.erf(x * 0.7071067811865476))


def _moe_block_kernel(task_ids_ref, task_embed_ref, gate_wx_ref, gate_wt_ref,
                      gate_b_ref, we_ref, be_ref, wu_ref, bu_ref, x_ref,
                      out_ref, *, blk, n_per_batch, num_experts):
    i = pl.program_id(0)
    b = (i * blk) // n_per_batch

    x = x_ref[...]

    t_id = task_ids_ref[0, b]
    t_vec = task_embed_ref[pl.ds(t_id, 1), :]
    dn = (((1,), (1,)), ((), ()))
    off = jax.lax.dot_general(t_vec, gate_wt_ref[...], dn,
                              preferred_element_type=jnp.float32)
    off = off + gate_b_ref[...]
    logits = jax.lax.dot_general(x, gate_wx_ref[...], dn,
                                 preferred_element_type=jnp.float32) + off

    e_iota = jax.lax.broadcasted_iota(jnp.int32, (blk, num_experts), 1)
    idx1 = jnp.argmax(logits, axis=1)
    m1 = jnp.max(logits, axis=1)
    eq1 = e_iota == idx1[:, None]
    l2 = jnp.where(eq1, -jnp.inf, logits)
    idx2 = jnp.argmax(l2, axis=1)
    m2 = jnp.max(l2, axis=1)
    denom = 1.0 + jnp.exp(m2 - m1)
    sel = eq1 | (e_iota == idx2[:, None])
    gates = jnp.where(sel, jnp.exp(logits - m1[:, None]), 0.0) / denom[:, None]
    omega = 1.0 - 1.0 / denom

    xb = x.astype(jnp.bfloat16)
    u = jax.lax.dot_general(xb, wu_ref[...], dn,
                            preferred_element_type=jnp.float32)
    u = _gelu_exact(u + bu_ref[...])
    acc = omega[:, None] * u
    for e in range(num_experts):
        h = jax.lax.dot_general(xb, we_ref[e], dn,
                                preferred_element_type=jnp.float32)
        h = _gelu_exact(h + be_ref[pl.ds(e, 1), :])
        acc = acc + gates[:, e][:, None] * h
    out_ref[...] = acc


def kernel(tokens, task_ids, task_embed, gate_W, gate_b, We, be, Wu, bu):
    B, N, D = tokens.shape
    E = gate_W.shape[0]
    blk = 256
    x2d = tokens.reshape(B * N, D)
    grid = (B * N // blk,)

    gate_wx = gate_W[:, :D]
    gate_wt = gate_W[:, D:]
    we_bf = We.astype(jnp.bfloat16)
    wu_bf = Wu.astype(jnp.bfloat16)

    full = lambda shape: pl.BlockSpec(shape, lambda i: (0,) * len(shape))
    out = pl.pallas_call(
        functools.partial(_moe_block_kernel, blk=blk, n_per_batch=N,
                          num_experts=E),
        grid=grid,
        in_specs=[
            pl.BlockSpec(memory_space=pltpu.SMEM),
            full(task_embed.shape),
            full(gate_wx.shape),
            full(gate_wt.shape),
            full((1, E)),
            full(we_bf.shape),
            full(be.shape),
            full(wu_bf.shape),
            full((1, D)),
            pl.BlockSpec((blk, D), lambda i: (i, 0)),
        ],
        out_specs=pl.BlockSpec((blk, D), lambda i: (i, 0)),
        out_shape=jax.ShapeDtypeStruct((B * N, D), jnp.float32),
        compiler_params=pltpu.CompilerParams(
            dimension_semantics=("arbitrary",),
        ),
    )(task_ids.reshape(1, B).astype(jnp.int32), task_embed, gate_wx, gate_wt,
      gate_b.reshape(1, E), we_bf, be, wu_bf, bu.reshape(1, D), x2d)
    return out.reshape(B, N, D)


# speedup vs baseline: 6.2696x; 1.0036x over previous
import functools

import jax
import jax.numpy as jnp
from jax.experimental import pallas as pl
from jax.experimental.pallas import tpu as pltpu


def _gelu_exact(x):
    return 0.5 * x * (1.0 + jax.lax.erf(x * 0.7071067811865476))


def _moe_block_kernel(task_ids_ref, task_embed_ref, gate_wx_ref, gate_wt_ref,
                      gate_b_ref, we_ref, be_ref, wu_ref, bu_ref, x_ref,
                      out_ref, *, blk, n_per_batch, num_experts):
    i = pl.program_id(0)
    b = (i * blk) // n_per_batch

    x = x_ref[...]

    t_id = task_ids_ref[0, b]
    t_vec = task_embed_ref[pl.ds(t_id, 1), :]
    dn = (((1,), (1,)), ((), ()))
    off = jax.lax.dot_general(t_vec, gate_wt_ref[...], dn,
                              preferred_element_type=jnp.float32)
    off = off + gate_b_ref[...]
    logits = jax.lax.dot_general(x, gate_wx_ref[...], dn,
                                 preferred_element_type=jnp.float32) + off

    e_iota = jax.lax.broadcasted_iota(jnp.int32, (blk, num_experts), 1)
    idx1 = jnp.argmax(logits, axis=1)
    m1 = jnp.max(logits, axis=1)
    eq1 = e_iota == idx1[:, None]
    l2 = jnp.where(eq1, -jnp.inf, logits)
    idx2 = jnp.argmax(l2, axis=1)
    m2 = jnp.max(l2, axis=1)
    denom = 1.0 + jnp.exp(m2 - m1)
    sel = eq1 | (e_iota == idx2[:, None])
    gates = jnp.where(sel, jnp.exp(logits - m1[:, None]), 0.0) / denom[:, None]
    omega = 1.0 - 1.0 / denom

    xb = x.astype(jnp.bfloat16)
    u = jax.lax.dot_general(xb, wu_ref[...], dn,
                            preferred_element_type=jnp.float32)
    u = _gelu_exact(u + bu_ref[...])
    acc = omega[:, None] * u
    for e in range(num_experts):
        h = jax.lax.dot_general(xb, we_ref[e], dn,
                                preferred_element_type=jnp.float32)
        h = _gelu_exact(h + be_ref[pl.ds(e, 1), :])
        acc = acc + gates[:, e][:, None] * h
    out_ref[...] = acc


def kernel(tokens, task_ids, task_embed, gate_W, gate_b, We, be, Wu, bu):
    B, N, D = tokens.shape
    E = gate_W.shape[0]
    blk = 512
    x2d = tokens.reshape(B * N, D)
    grid = (B * N // blk,)

    gate_wx = gate_W[:, :D]
    gate_wt = gate_W[:, D:]
    we_bf = We.astype(jnp.bfloat16)
    wu_bf = Wu.astype(jnp.bfloat16)

    full = lambda shape: pl.BlockSpec(shape, lambda i: (0,) * len(shape))
    out = pl.pallas_call(
        functools.partial(_moe_block_kernel, blk=blk, n_per_batch=N,
                          num_experts=E),
        grid=grid,
        in_specs=[
            pl.BlockSpec(memory_space=pltpu.SMEM),
            full(task_embed.shape),
            full(gate_wx.shape),
            full(gate_wt.shape),
            full((1, E)),
            full(we_bf.shape),
            full(be.shape),
            full(wu_bf.shape),
            full((1, D)),
            pl.BlockSpec((blk, D), lambda i: (i, 0)),
        ],
        out_specs=pl.BlockSpec((blk, D), lambda i: (i, 0)),
        out_shape=jax.ShapeDtypeStruct((B * N, D), jnp.float32),
        compiler_params=pltpu.CompilerParams(
            dimension_semantics=("arbitrary",),
        ),
    )(task_ids.reshape(1, B).astype(jnp.int32), task_embed, gate_wx, gate_wt,
      gate_b.reshape(1, E), we_bf, be, wu_bf, bu.reshape(1, D), x2d)
    return out.reshape(B, N, D)
